# baseline (speedup 1.0000x reference)
import jax
import jax.numpy as jnp
from jax.experimental import pallas as pl
from jax.experimental.pallas import tpu as pltpu

_B = 16384
_RB = 512
_W = 256


def _body(x_ref, o_ref):
    x = x_ref[...]
    terms = []
    for p in range(16):
        gmax = jnp.max(x[:, 16 + 8 * p: 24 + 8 * p], axis=1)
        terms.append(jnp.abs(x[:, p] - gmax))
    s = jnp.sum(jnp.stack(terms))

    @pl.when(pl.program_id(0) == 0)
    def _():
        o_ref[0, 0] = 0.0

    o_ref[0, 0] += s


def kernel(preds):
    out = pl.pallas_call(
        _body,
        grid=(_B // _RB,),
        in_specs=[pl.BlockSpec((_RB, _W), lambda i: (i, 0))],
        out_specs=pl.BlockSpec(memory_space=pltpu.SMEM),
        out_shape=jax.ShapeDtypeStruct((1, 1), jnp.float32),
    )(preds)
    return out[0, 0] / preds.shape[0]
